# Initial kernel scaffold; baseline (speedup 1.0000x reference)
import jax
import jax.numpy as jnp
from jax.experimental import pallas as pl


def kernel(x, edge_index, edge_weights, edge_attr, pre_g, pre_b, ln_g, ln_b, b0_ng, b0_nb, b0_Wl, b0_bl, b0_Wr, b1_ng, b1_nb, b1_Wl, b1_bl, b1_Wr, et_ng, et_nb, et_W, et_b, en_g, en_b):
    raise NotImplementedError("write your pallas kernel here")


# speedup vs baseline: 4.5553x; 4.5553x over previous
import functools

import jax
import jax.numpy as jnp
from jax import lax
from jax.experimental import pallas as pl
from jax.experimental.pallas import tpu as pltpu
from jax.experimental.pallas import tpu_sc as plsc

_N = 10000
_E = 320000
_D = 128
_DG = 64
_DGC = 80
_DE = 16
_EPS = 1e-5

_NC = 2
_NS = 16
_EPW = _E // (_NC * _NS)
_C = 80
_NCHUNK = _EPW // _C
_RPT = 624
_RPT_LAST = _N - _RPT * (_NS - 1)


def _ln(v, g, b):
    mu = jnp.mean(v, axis=-1, keepdims=True)
    d = v - mu
    var = jnp.mean(d * d, axis=-1, keepdims=True)
    return d * lax.rsqrt(var + _EPS) * g + b


def _gelu(v):
    return 0.5 * v * (1.0 + lax.erf(v * 0.7071067811865476))


def _seg_body(width, h_hbm, src_hbm, dst_hbm, agg_out,
              sidx, didx, rows, zbuf, acc_sh, gsem):
    c = lax.axis_index("c")
    s = lax.axis_index("s")

    def _z0(i, _):
        for j in range(width // 16):
            zbuf[i, pl.ds(j * 16, 16)] = jnp.zeros((16,), jnp.float32)
        return 0
    lax.fori_loop(0, _RPT_LAST, _z0, 0)

    @pl.when(s < _NS - 1)
    def _():
        pltpu.sync_copy(zbuf.at[pl.ds(0, _RPT)], acc_sh.at[pl.ds(s * _RPT, _RPT)])

    @pl.when(s == _NS - 1)
    def _():
        pltpu.sync_copy(zbuf, acc_sh.at[pl.ds((_NS - 1) * _RPT, _RPT_LAST)])

    plsc.subcore_barrier()

    base = (c * _NS + s) * _EPW

    def _step(k, _):
        eb = base + k * _C
        pltpu.sync_copy(src_hbm.at[pl.ds(eb, _C)], sidx)
        pltpu.sync_copy(dst_hbm.at[pl.ds(eb, _C)], didx)
        pltpu.async_copy(h_hbm.at[sidx], rows, gsem).wait()
        pltpu.sync_copy(rows, acc_sh.at[didx], add=True)
        return 0

    lax.fori_loop(0, _NCHUNK, _step, 0)

    plsc.subcore_barrier()

    @pl.when(s < _NS - 1)
    def _():
        pltpu.sync_copy(acc_sh.at[pl.ds(s * _RPT, _RPT)], zbuf.at[pl.ds(0, _RPT)])
        pltpu.sync_copy(zbuf.at[pl.ds(0, _RPT)], agg_out.at[c, pl.ds(s * _RPT, _RPT)])

    @pl.when(s == _NS - 1)
    def _():
        pltpu.sync_copy(acc_sh.at[pl.ds((_NS - 1) * _RPT, _RPT_LAST)], zbuf)
        pltpu.sync_copy(zbuf, agg_out.at[c, pl.ds((_NS - 1) * _RPT, _RPT_LAST)])


def _make_seg(width):
    mesh = plsc.VectorSubcoreMesh(
        core_axis_name="c", subcore_axis_name="s",
        num_cores=_NC, num_subcores=_NS)
    return pl.kernel(
        functools.partial(_seg_body, width),
        out_type=jax.ShapeDtypeStruct((_NC, _N, width), jnp.float32),
        mesh=mesh,
        compiler_params=pltpu.CompilerParams(use_tc_tiling_on_sc=False),
        scratch_types=[
            pltpu.VMEM((_C,), jnp.int32),
            pltpu.VMEM((_C,), jnp.int32),
            pltpu.VMEM((_C, width), jnp.float32),
            pltpu.VMEM((_RPT_LAST, width), jnp.float32),
            pltpu.VMEM_SHARED((_N, width), jnp.float32),
            pltpu.SemaphoreType.DMA,
        ],
    )


_RB = 1000
_NRB = _N // _RB


def _pre_body(x_ref, pg, pb, ng, nb, wr, bl,
              x2_ref, h2t_ref, y1b_ref):
    h = _gelu(_ln(x_ref[...], pg[...], pb[...]))
    x1 = h[:, :_DG]
    x2 = h[:, _DG:]
    h2 = jax.nn.relu(_ln(x2, ng[...], nb[...]))
    x2_ref[...] = x2
    h2t_ref[...] = jnp.concatenate(
        [h2,
         jnp.ones((_RB, 1), jnp.float32),
         jnp.zeros((_RB, _DGC - _DG - 1), jnp.float32)], axis=1)
    y1b_ref[...] = x1 + jnp.dot(h2, wr[...].T,
                                preferred_element_type=jnp.float32) + bl[...]


def _mid_body(aggp, y1b, x2_ref, wl, ng, nb, wr, bl,
              y1_ref, g1_ref, y2b_ref, inv_ref):
    ap = aggp[0] + aggp[1]
    cnt = ap[:, _DG:_DG + 1]
    inv = 1.0 / jnp.maximum(cnt, 1.0)
    agg = ap[:, :_DG] * inv
    y1 = y1b[...] + jnp.dot(agg, wl[...].T,
                            preferred_element_type=jnp.float32)
    g1 = jax.nn.relu(_ln(y1, ng[...], nb[...]))
    y1_ref[...] = y1
    g1_ref[...] = g1
    y2b_ref[...] = x2_ref[...] + jnp.dot(
        g1, wr[...].T, preferred_element_type=jnp.float32) + bl[...]
    inv_ref[...] = inv


def _post_body(aggp, inv_ref, y1_ref, y2b, xres, wl, lg, lb, out_ref):
    agg = (aggp[0] + aggp[1]) * inv_ref[...]
    y2 = y2b[...] + jnp.dot(agg, wl[...].T,
                            preferred_element_type=jnp.float32)
    h = jnp.concatenate([y1_ref[...], y2], axis=1) + xres[...]
    out_ref[...] = _ln(h, lg[...], lb[...])


_EB = 2000
_NEB = (_E // 8) // _EB


def _edge_body(ea, m_ref, wb_ref, g1, b1, bt, g2, b2, out_ref):
    a = ea[...]
    m = m_ref[...]
    mu = jnp.dot(a, m, preferred_element_type=jnp.float32)
    d = a - mu
    var = jnp.dot(d * d, m, preferred_element_type=jnp.float32)
    h = _gelu(d * lax.rsqrt(var + _EPS) * g1[...] + b1[...])
    e = jnp.dot(h, wb_ref[...], preferred_element_type=jnp.float32) + bt[...] + a
    mu2 = jnp.dot(e, m, preferred_element_type=jnp.float32)
    d2 = e - mu2
    var2 = jnp.dot(d2 * d2, m, preferred_element_type=jnp.float32)
    out_ref[...] = d2 * lax.rsqrt(var2 + _EPS) * g2[...] + b2[...]


def _vec_spec(n):
    return pl.BlockSpec((n,), lambda i: (0,))


def _mat_spec(r, c):
    return pl.BlockSpec((r, c), lambda i: (0, 0))


def kernel(x, edge_index, edge_weights, edge_attr, pre_g, pre_b, ln_g, ln_b,
           b0_ng, b0_nb, b0_Wl, b0_bl, b0_Wr, b1_ng, b1_nb, b1_Wl, b1_bl,
           b1_Wr, et_ng, et_nb, et_W, et_b, en_g, en_b):
    src = edge_index[0]
    dst = edge_index[1]

    x2, h2t, y1b = pl.pallas_call(
        _pre_body,
        grid=(_NRB,),
        in_specs=[
            pl.BlockSpec((_RB, _D), lambda i: (i, 0)),
            _vec_spec(_D), _vec_spec(_D),
            _vec_spec(_DG), _vec_spec(_DG),
            _mat_spec(_DG, _DG), _vec_spec(_DG),
        ],
        out_specs=[
            pl.BlockSpec((_RB, _DG), lambda i: (i, 0)),
            pl.BlockSpec((_RB, _DGC), lambda i: (i, 0)),
            pl.BlockSpec((_RB, _DG), lambda i: (i, 0)),
        ],
        out_shape=[
            jax.ShapeDtypeStruct((_N, _DG), jnp.float32),
            jax.ShapeDtypeStruct((_N, _DGC), jnp.float32),
            jax.ShapeDtypeStruct((_N, _DG), jnp.float32),
        ],
    )(x, pre_g, pre_b, b0_ng, b0_nb, b0_Wr, b0_bl)

    agg0_p = _make_seg(_DGC)(h2t, src, dst)

    y1, g1, y2b, inv = pl.pallas_call(
        _mid_body,
        grid=(_NRB,),
        in_specs=[
            pl.BlockSpec((_NC, _RB, _DGC), lambda i: (0, i, 0)),
            pl.BlockSpec((_RB, _DG), lambda i: (i, 0)),
            pl.BlockSpec((_RB, _DG), lambda i: (i, 0)),
            _mat_spec(_DG, _DG),
            _vec_spec(_DG), _vec_spec(_DG),
            _mat_spec(_DG, _DG), _vec_spec(_DG),
        ],
        out_specs=[
            pl.BlockSpec((_RB, _DG), lambda i: (i, 0)),
            pl.BlockSpec((_RB, _DG), lambda i: (i, 0)),
            pl.BlockSpec((_RB, _DG), lambda i: (i, 0)),
            pl.BlockSpec((_RB, 1), lambda i: (i, 0)),
        ],
        out_shape=[
            jax.ShapeDtypeStruct((_N, _DG), jnp.float32),
            jax.ShapeDtypeStruct((_N, _DG), jnp.float32),
            jax.ShapeDtypeStruct((_N, _DG), jnp.float32),
            jax.ShapeDtypeStruct((_N, 1), jnp.float32),
        ],
    )(agg0_p, y1b, x2, b0_Wl, b1_ng, b1_nb, b1_Wr, b1_bl)

    agg1_p = _make_seg(_DG)(g1, src, dst)

    x_out = pl.pallas_call(
        _post_body,
        grid=(_NRB,),
        in_specs=[
            pl.BlockSpec((_NC, _RB, _DG), lambda i: (0, i, 0)),
            pl.BlockSpec((_RB, 1), lambda i: (i, 0)),
            pl.BlockSpec((_RB, _DG), lambda i: (i, 0)),
            pl.BlockSpec((_RB, _DG), lambda i: (i, 0)),
            pl.BlockSpec((_RB, _D), lambda i: (i, 0)),
            _mat_spec(_DG, _DG),
            _vec_spec(_D), _vec_spec(_D),
        ],
        out_specs=pl.BlockSpec((_RB, _D), lambda i: (i, 0)),
        out_shape=jax.ShapeDtypeStruct((_N, _D), jnp.float32),
    )(agg1_p, inv, y1, y2b, x, b1_Wl, ln_g, ln_b)

    eye8 = jnp.eye(8, dtype=jnp.float32)
    mmat = jnp.kron(eye8, jnp.full((_DE, _DE), 1.0 / _DE, jnp.float32))
    wbig = jnp.kron(eye8, et_W.T)
    ea2 = edge_attr.reshape(_E // 8, _D)
    e2 = pl.pallas_call(
        _edge_body,
        grid=(_NEB,),
        in_specs=[
            pl.BlockSpec((_EB, _D), lambda i: (i, 0)),
            _mat_spec(_D, _D), _mat_spec(_D, _D),
            _vec_spec(_D), _vec_spec(_D), _vec_spec(_D),
            _vec_spec(_D), _vec_spec(_D),
        ],
        out_specs=pl.BlockSpec((_EB, _D), lambda i: (i, 0)),
        out_shape=jax.ShapeDtypeStruct((_E // 8, _D), jnp.float32),
    )(ea2, mmat, wbig, jnp.tile(et_ng, 8), jnp.tile(et_nb, 8),
      jnp.tile(et_b, 8), jnp.tile(en_g, 8), jnp.tile(en_b, 8))
    e_out = e2.reshape(_E, _DE)

    return (x_out, edge_index, edge_weights, e_out)


# speedup vs baseline: 6.6372x; 1.4570x over previous
import functools

import jax
import jax.numpy as jnp
from jax import lax
from jax.experimental import pallas as pl
from jax.experimental.pallas import tpu as pltpu
from jax.experimental.pallas import tpu_sc as plsc

_N = 10000
_E = 320000
_D = 128
_DG = 64
_DE = 16
_EPS = 1e-5

_NC = 2
_NS = 16
_EPW = _E // (_NC * _NS)
_C = 80
_NCHUNK = _EPW // _C
_RPT = 624
_RPT_LAST = _N - _RPT * (_NS - 1)


def _ln(v, g, b):
    mu = jnp.mean(v, axis=-1, keepdims=True)
    d = v - mu
    var = jnp.mean(d * d, axis=-1, keepdims=True)
    return d * lax.rsqrt(var + _EPS) * g + b


def _gelu(v):
    return 0.5 * v * (1.0 + lax.erf(v * 0.7071067811865476))


def _seg_body(with_count, *refs):
    if with_count:
        (h_hbm, src_hbm, dst_hbm, agg_out, cnt_out,
         sidx, didx, rows, zbuf, ones_v, cbuf, acc_sh, cnt_sh,
         gsem0, gsem1) = refs
    else:
        (h_hbm, src_hbm, dst_hbm, agg_out,
         sidx, didx, rows, zbuf, acc_sh, gsem0, gsem1) = refs
    c = lax.axis_index("c")
    s = lax.axis_index("s")

    def _z0(i, _):
        for j in range(_DG // 16):
            zbuf[i, pl.ds(j * 16, 16)] = jnp.zeros((16,), jnp.float32)
        return 0
    lax.fori_loop(0, _RPT_LAST, _z0, 0)
    if with_count:
        for j in range(_RPT_LAST // 16):
            cbuf[pl.ds(j * 16, 16)] = jnp.zeros((16,), jnp.float32)
        for j in range(_C // 16):
            ones_v[pl.ds(j * 16, 16)] = jnp.ones((16,), jnp.float32)

    @pl.when(s < _NS - 1)
    def _():
        pltpu.sync_copy(zbuf.at[pl.ds(0, _RPT)], acc_sh.at[pl.ds(s * _RPT, _RPT)])
        if with_count:
            pltpu.sync_copy(cbuf.at[pl.ds(0, _RPT)],
                            cnt_sh.at[pl.ds(s * _RPT, _RPT)])

    @pl.when(s == _NS - 1)
    def _():
        pltpu.sync_copy(zbuf, acc_sh.at[pl.ds((_NS - 1) * _RPT, _RPT_LAST)])
        if with_count:
            pltpu.sync_copy(cbuf, cnt_sh.at[pl.ds((_NS - 1) * _RPT, _RPT_LAST)])

    plsc.subcore_barrier()

    ebase = (c * _NS + s) * _EPW
    pltpu.sync_copy(src_hbm.at[pl.ds(ebase, _EPW)], sidx)
    pltpu.sync_copy(dst_hbm.at[pl.ds(ebase, _EPW)], didx)

    def _sl(ref, k):
        return ref.at[pl.ds(k * _C, _C)]

    def _scat(k, b):
        pltpu.sync_copy(rows.at[b], acc_sh.at[_sl(didx, k)], add=True)
        if with_count:
            pltpu.sync_copy(ones_v, cnt_sh.at[_sl(didx, k)], add=True)

    pltpu.async_copy(h_hbm.at[_sl(sidx, 0)], rows.at[0], gsem0)

    def _wait(k, b, sem):
        pltpu.make_async_copy(h_hbm.at[_sl(sidx, k)], rows.at[b], sem).wait()

    def _pair(g, _):
        k0 = 2 * g
        pltpu.async_copy(h_hbm.at[_sl(sidx, k0 + 1)], rows.at[1], gsem1)
        _wait(k0, 0, gsem0)
        _scat(k0, 0)
        pltpu.async_copy(h_hbm.at[_sl(sidx, k0 + 2)], rows.at[0], gsem0)
        _wait(k0 + 1, 1, gsem1)
        _scat(k0 + 1, 1)
        return 0

    lax.fori_loop(0, (_NCHUNK - 1) // 2, _pair, 0)
    _wait(_NCHUNK - 1, 0, gsem0)
    _scat(_NCHUNK - 1, 0)

    plsc.subcore_barrier()

    @pl.when(s < _NS - 1)
    def _():
        pltpu.sync_copy(acc_sh.at[pl.ds(s * _RPT, _RPT)], zbuf.at[pl.ds(0, _RPT)])
        pltpu.sync_copy(zbuf.at[pl.ds(0, _RPT)], agg_out.at[c, pl.ds(s * _RPT, _RPT)])
        if with_count:
            pltpu.sync_copy(cnt_sh.at[pl.ds(s * _RPT, _RPT)],
                            cbuf.at[pl.ds(0, _RPT)])
            pltpu.sync_copy(cbuf.at[pl.ds(0, _RPT)],
                            cnt_out.at[c, pl.ds(s * _RPT, _RPT)])

    @pl.when(s == _NS - 1)
    def _():
        pltpu.sync_copy(acc_sh.at[pl.ds((_NS - 1) * _RPT, _RPT_LAST)], zbuf)
        pltpu.sync_copy(zbuf, agg_out.at[c, pl.ds((_NS - 1) * _RPT, _RPT_LAST)])
        if with_count:
            pltpu.sync_copy(cnt_sh.at[pl.ds((_NS - 1) * _RPT, _RPT_LAST)], cbuf)
            pltpu.sync_copy(cbuf, cnt_out.at[c, pl.ds((_NS - 1) * _RPT, _RPT_LAST)])


def _make_seg(with_count):
    mesh = plsc.VectorSubcoreMesh(
        core_axis_name="c", subcore_axis_name="s",
        num_cores=_NC, num_subcores=_NS)
    out_type = [jax.ShapeDtypeStruct((_NC, _N, _DG), jnp.float32)]
    scratch = [
        pltpu.VMEM((_EPW,), jnp.int32),
        pltpu.VMEM((_EPW,), jnp.int32),
        pltpu.VMEM((2, _C, _DG), jnp.float32),
        pltpu.VMEM((_RPT_LAST, _DG), jnp.float32),
    ]
    if with_count:
        out_type.append(jax.ShapeDtypeStruct((_NC, _N), jnp.float32))
        scratch.append(pltpu.VMEM((_C,), jnp.float32))
        scratch.append(pltpu.VMEM((_RPT_LAST,), jnp.float32))
    scratch.append(pltpu.VMEM_SHARED((_N, _DG), jnp.float32))
    if with_count:
        scratch.append(pltpu.VMEM_SHARED((_N,), jnp.float32))
    scratch.append(pltpu.SemaphoreType.DMA)
    scratch.append(pltpu.SemaphoreType.DMA)
    return pl.kernel(
        functools.partial(_seg_body, with_count),
        out_type=tuple(out_type) if with_count else out_type[0],
        mesh=mesh,
        compiler_params=pltpu.CompilerParams(use_tc_tiling_on_sc=False),
        scratch_types=scratch,
    )


_RB = 1000
_NRB = _N // _RB


def _pre_body(x_ref, pg, pb, ng, nb, wr, bl,
              x2_ref, h2_ref, y1b_ref):
    h = _gelu(_ln(x_ref[...], pg[...], pb[...]))
    x1 = h[:, :_DG]
    x2 = h[:, _DG:]
    h2 = jax.nn.relu(_ln(x2, ng[...], nb[...]))
    x2_ref[...] = x2
    h2_ref[...] = h2
    y1b_ref[...] = x1 + jnp.dot(h2, wr[...].T,
                                preferred_element_type=jnp.float32) + bl[...]


def _mid_body(aggp, cntp, y1b, x2_ref, wl, ng, nb, wr, bl,
              y1_ref, g1_ref, y2b_ref, inv_ref):
    cnt = cntp[0] + cntp[1]
    inv = 1.0 / jnp.maximum(cnt, 1.0)
    agg = (aggp[0] + aggp[1]) * inv
    y1 = y1b[...] + jnp.dot(agg, wl[...].T,
                            preferred_element_type=jnp.float32)
    g1 = jax.nn.relu(_ln(y1, ng[...], nb[...]))
    y1_ref[...] = y1
    g1_ref[...] = g1
    y2b_ref[...] = x2_ref[...] + jnp.dot(
        g1, wr[...].T, preferred_element_type=jnp.float32) + bl[...]
    inv_ref[...] = inv


def _post_body(aggp, inv_ref, y1_ref, y2b, xres, wl, lg, lb, out_ref):
    agg = (aggp[0] + aggp[1]) * inv_ref[...]
    y2 = y2b[...] + jnp.dot(agg, wl[...].T,
                            preferred_element_type=jnp.float32)
    h = jnp.concatenate([y1_ref[...], y2], axis=1) + xres[...]
    out_ref[...] = _ln(h, lg[...], lb[...])


_EB = 2000
_NEB = (_E // 8) // _EB


def _edge_body(ea, m_ref, wb_ref, g1, b1, bt, g2, b2, out_ref):
    a = ea[...]
    m = m_ref[...]
    mu = jnp.dot(a, m, preferred_element_type=jnp.float32)
    d = a - mu
    var = jnp.dot(d * d, m, preferred_element_type=jnp.float32)
    h = _gelu(d * lax.rsqrt(var + _EPS) * g1[...] + b1[...])
    e = jnp.dot(h, wb_ref[...], preferred_element_type=jnp.float32) + bt[...] + a
    mu2 = jnp.dot(e, m, preferred_element_type=jnp.float32)
    d2 = e - mu2
    var2 = jnp.dot(d2 * d2, m, preferred_element_type=jnp.float32)
    out_ref[...] = d2 * lax.rsqrt(var2 + _EPS) * g2[...] + b2[...]


def _vec_spec(n):
    return pl.BlockSpec((n,), lambda i: (0,))


def _mat_spec(r, c):
    return pl.BlockSpec((r, c), lambda i: (0, 0))


def kernel(x, edge_index, edge_weights, edge_attr, pre_g, pre_b, ln_g, ln_b,
           b0_ng, b0_nb, b0_Wl, b0_bl, b0_Wr, b1_ng, b1_nb, b1_Wl, b1_bl,
           b1_Wr, et_ng, et_nb, et_W, et_b, en_g, en_b):
    src = edge_index[0]
    dst = edge_index[1]

    x2, h2, y1b = pl.pallas_call(
        _pre_body,
        grid=(_NRB,),
        in_specs=[
            pl.BlockSpec((_RB, _D), lambda i: (i, 0)),
            _vec_spec(_D), _vec_spec(_D),
            _vec_spec(_DG), _vec_spec(_DG),
            _mat_spec(_DG, _DG), _vec_spec(_DG),
        ],
        out_specs=[
            pl.BlockSpec((_RB, _DG), lambda i: (i, 0)),
            pl.BlockSpec((_RB, _DG), lambda i: (i, 0)),
            pl.BlockSpec((_RB, _DG), lambda i: (i, 0)),
        ],
        out_shape=[
            jax.ShapeDtypeStruct((_N, _DG), jnp.float32),
            jax.ShapeDtypeStruct((_N, _DG), jnp.float32),
            jax.ShapeDtypeStruct((_N, _DG), jnp.float32),
        ],
    )(x, pre_g, pre_b, b0_ng, b0_nb, b0_Wr, b0_bl)

    agg0_p, cnt_p = _make_seg(True)(h2, src, dst)
    cnt3 = cnt_p.reshape(_NC, _N, 1)

    y1, g1, y2b, inv = pl.pallas_call(
        _mid_body,
        grid=(_NRB,),
        in_specs=[
            pl.BlockSpec((_NC, _RB, _DG), lambda i: (0, i, 0)),
            pl.BlockSpec((_NC, _RB, 1), lambda i: (0, i, 0)),
            pl.BlockSpec((_RB, _DG), lambda i: (i, 0)),
            pl.BlockSpec((_RB, _DG), lambda i: (i, 0)),
            _mat_spec(_DG, _DG),
            _vec_spec(_DG), _vec_spec(_DG),
            _mat_spec(_DG, _DG), _vec_spec(_DG),
        ],
        out_specs=[
            pl.BlockSpec((_RB, _DG), lambda i: (i, 0)),
            pl.BlockSpec((_RB, _DG), lambda i: (i, 0)),
            pl.BlockSpec((_RB, _DG), lambda i: (i, 0)),
            pl.BlockSpec((_RB, 1), lambda i: (i, 0)),
        ],
        out_shape=[
            jax.ShapeDtypeStruct((_N, _DG), jnp.float32),
            jax.ShapeDtypeStruct((_N, _DG), jnp.float32),
            jax.ShapeDtypeStruct((_N, _DG), jnp.float32),
            jax.ShapeDtypeStruct((_N, 1), jnp.float32),
        ],
    )(agg0_p, cnt3, y1b, x2, b0_Wl, b1_ng, b1_nb, b1_Wr, b1_bl)

    agg1_p = _make_seg(False)(g1, src, dst)

    x_out = pl.pallas_call(
        _post_body,
        grid=(_NRB,),
        in_specs=[
            pl.BlockSpec((_NC, _RB, _DG), lambda i: (0, i, 0)),
            pl.BlockSpec((_RB, 1), lambda i: (i, 0)),
            pl.BlockSpec((_RB, _DG), lambda i: (i, 0)),
            pl.BlockSpec((_RB, _DG), lambda i: (i, 0)),
            pl.BlockSpec((_RB, _D), lambda i: (i, 0)),
            _mat_spec(_DG, _DG),
            _vec_spec(_D), _vec_spec(_D),
        ],
        out_specs=pl.BlockSpec((_RB, _D), lambda i: (i, 0)),
        out_shape=jax.ShapeDtypeStruct((_N, _D), jnp.float32),
    )(agg1_p, inv, y1, y2b, x, b1_Wl, ln_g, ln_b)

    eye8 = jnp.eye(8, dtype=jnp.float32)
    mmat = jnp.kron(eye8, jnp.full((_DE, _DE), 1.0 / _DE, jnp.float32))
    wbig = jnp.kron(eye8, et_W.T)
    ea2 = edge_attr.reshape(_E // 8, _D)
    e2 = pl.pallas_call(
        _edge_body,
        grid=(_NEB,),
        in_specs=[
            pl.BlockSpec((_EB, _D), lambda i: (i, 0)),
            _mat_spec(_D, _D), _mat_spec(_D, _D),
            _vec_spec(_D), _vec_spec(_D), _vec_spec(_D),
            _vec_spec(_D), _vec_spec(_D),
        ],
        out_specs=pl.BlockSpec((_EB, _D), lambda i: (i, 0)),
        out_shape=jax.ShapeDtypeStruct((_E // 8, _D), jnp.float32),
    )(ea2, mmat, wbig, jnp.tile(et_ng, 8), jnp.tile(et_nb, 8),
      jnp.tile(et_b, 8), jnp.tile(en_g, 8), jnp.tile(en_b, 8))
    e_out = e2.reshape(_E, _DE)

    return (x_out, edge_index, edge_weights, e_out)


# speedup vs baseline: 6.7261x; 1.0134x over previous
import functools

import jax
import jax.numpy as jnp
from jax import lax
from jax.experimental import pallas as pl
from jax.experimental.pallas import tpu as pltpu
from jax.experimental.pallas import tpu_sc as plsc

_N = 10000
_E = 320000
_D = 128
_DG = 64
_DE = 16
_EPS = 1e-5

_NC = 2
_NS = 16
_EPW = _E // (_NC * _NS)
_C = 80
_NCHUNK = _EPW // _C
_RPT = 624
_RPT_LAST = _N - _RPT * (_NS - 1)


def _ln(v, g, b):
    mu = jnp.mean(v, axis=-1, keepdims=True)
    d = v - mu
    var = jnp.mean(d * d, axis=-1, keepdims=True)
    return d * lax.rsqrt(var + _EPS) * g + b


def _gelu(v):
    return 0.5 * v * (1.0 + lax.erf(v * 0.7071067811865476))


def _seg_body(with_count, *refs):
    if with_count:
        (h_hbm, src_hbm, dst_hbm, agg_out, cnt_out,
         sidx, didx, rows, zbuf, ones_v, cbuf, acc_sh, cnt_sh,
         gsem0, gsem1) = refs
    else:
        (h_hbm, src_hbm, dst_hbm, agg_out,
         sidx, didx, rows, zbuf, acc_sh, gsem0, gsem1) = refs
    c = lax.axis_index("c")
    s = lax.axis_index("s")

    def _z0(i, _):
        for j in range(_DG // 16):
            zbuf[i, pl.ds(j * 16, 16)] = jnp.zeros((16,), jnp.float32)
        return 0
    lax.fori_loop(0, _RPT_LAST, _z0, 0)
    if with_count:
        for j in range(_RPT_LAST // 16):
            cbuf[pl.ds(j * 16, 16)] = jnp.zeros((16,), jnp.float32)
        for j in range(_C // 16):
            ones_v[pl.ds(j * 16, 16)] = jnp.ones((16,), jnp.float32)

    @pl.when(s < _NS - 1)
    def _():
        pltpu.sync_copy(zbuf.at[pl.ds(0, _RPT)], acc_sh.at[pl.ds(s * _RPT, _RPT)])
        if with_count:
            pltpu.sync_copy(cbuf.at[pl.ds(0, _RPT)],
                            cnt_sh.at[pl.ds(s * _RPT, _RPT)])

    @pl.when(s == _NS - 1)
    def _():
        pltpu.sync_copy(zbuf, acc_sh.at[pl.ds((_NS - 1) * _RPT, _RPT_LAST)])
        if with_count:
            pltpu.sync_copy(cbuf, cnt_sh.at[pl.ds((_NS - 1) * _RPT, _RPT_LAST)])

    plsc.subcore_barrier()

    ebase = (c * _NS + s) * _EPW
    pltpu.sync_copy(src_hbm.at[pl.ds(ebase, _EPW)], sidx)
    pltpu.sync_copy(dst_hbm.at[pl.ds(ebase, _EPW)], didx)

    def _dbl(i, _):
        v = sidx[pl.ds(i * 16, 16)]
        sidx[pl.ds(i * 16, 16)] = v + v
        return 0
    lax.fori_loop(0, _EPW // 16, _dbl, 0)

    def _sl(ref, k):
        return ref.at[pl.ds(k * _C, _C)]

    def _scat(k, b):
        pltpu.sync_copy(rows.at[b], acc_sh.at[_sl(didx, k)], add=True)
        if with_count:
            pltpu.sync_copy(ones_v, cnt_sh.at[_sl(didx, k)], add=True)

    pltpu.async_copy(h_hbm.at[_sl(sidx, 0)], rows.at[0], gsem0)

    def _wait(k, b, sem):
        pltpu.make_async_copy(h_hbm.at[_sl(sidx, k)], rows.at[b], sem).wait()

    def _pair(g, _):
        k0 = 2 * g
        pltpu.async_copy(h_hbm.at[_sl(sidx, k0 + 1)], rows.at[1], gsem1)
        _wait(k0, 0, gsem0)
        _scat(k0, 0)
        pltpu.async_copy(h_hbm.at[_sl(sidx, k0 + 2)], rows.at[0], gsem0)
        _wait(k0 + 1, 1, gsem1)
        _scat(k0 + 1, 1)
        return 0

    lax.fori_loop(0, (_NCHUNK - 1) // 2, _pair, 0)
    _wait(_NCHUNK - 1, 0, gsem0)
    _scat(_NCHUNK - 1, 0)

    plsc.subcore_barrier()

    @pl.when(s < _NS - 1)
    def _():
        pltpu.sync_copy(acc_sh.at[pl.ds(s * _RPT, _RPT)], zbuf.at[pl.ds(0, _RPT)])
        pltpu.sync_copy(zbuf.at[pl.ds(0, _RPT)], agg_out.at[c, pl.ds(s * _RPT, _RPT)])
        if with_count:
            pltpu.sync_copy(cnt_sh.at[pl.ds(s * _RPT, _RPT)],
                            cbuf.at[pl.ds(0, _RPT)])
            pltpu.sync_copy(cbuf.at[pl.ds(0, _RPT)],
                            cnt_out.at[c, pl.ds(s * _RPT, _RPT)])

    @pl.when(s == _NS - 1)
    def _():
        pltpu.sync_copy(acc_sh.at[pl.ds((_NS - 1) * _RPT, _RPT_LAST)], zbuf)
        pltpu.sync_copy(zbuf, agg_out.at[c, pl.ds((_NS - 1) * _RPT, _RPT_LAST)])
        if with_count:
            pltpu.sync_copy(cnt_sh.at[pl.ds((_NS - 1) * _RPT, _RPT_LAST)], cbuf)
            pltpu.sync_copy(cbuf, cnt_out.at[c, pl.ds((_NS - 1) * _RPT, _RPT_LAST)])


def _make_seg(with_count):
    mesh = plsc.VectorSubcoreMesh(
        core_axis_name="c", subcore_axis_name="s",
        num_cores=_NC, num_subcores=_NS)
    out_type = [jax.ShapeDtypeStruct((_NC, _N, _DG), jnp.float32)]
    scratch = [
        pltpu.VMEM((_EPW,), jnp.int32),
        pltpu.VMEM((_EPW,), jnp.int32),
        pltpu.VMEM((2, _C, _DG), jnp.float32),
        pltpu.VMEM((_RPT_LAST, _DG), jnp.float32),
    ]
    if with_count:
        out_type.append(jax.ShapeDtypeStruct((_NC, _N), jnp.float32))
        scratch.append(pltpu.VMEM((_C,), jnp.float32))
        scratch.append(pltpu.VMEM((_RPT_LAST,), jnp.float32))
    scratch.append(pltpu.VMEM_SHARED((_N, _DG), jnp.float32))
    if with_count:
        scratch.append(pltpu.VMEM_SHARED((_N,), jnp.float32))
    scratch.append(pltpu.SemaphoreType.DMA)
    scratch.append(pltpu.SemaphoreType.DMA)
    return pl.kernel(
        functools.partial(_seg_body, with_count),
        out_type=tuple(out_type) if with_count else out_type[0],
        mesh=mesh,
        compiler_params=pltpu.CompilerParams(use_tc_tiling_on_sc=False),
        scratch_types=scratch,
    )


_RB = 1000
_NRB = _N // _RB


def _pre_body(x_ref, pg, pb, ng, nb, wr, bl,
              x2_ref, h2w_ref, y1b_ref):
    h = _gelu(_ln(x_ref[...], pg[...], pb[...]))
    x1 = h[:, :_DG]
    x2 = h[:, _DG:]
    h2 = jax.nn.relu(_ln(x2, ng[...], nb[...]))
    x2_ref[...] = x2
    h2w_ref[...] = jnp.concatenate(
        [h2, jnp.zeros((_RB, _D - _DG), jnp.float32)], axis=1)
    y1b_ref[...] = x1 + jnp.dot(h2, wr[...].T,
                                preferred_element_type=jnp.float32) + bl[...]


def _mid_body(aggp, cntp, y1b, x2_ref, wl, ng, nb, wr, bl,
              y1_ref, g1w_ref, y2b_ref, inv_ref):
    cnt = cntp[0] + cntp[1]
    inv = 1.0 / jnp.maximum(cnt, 1.0)
    agg = (aggp[0] + aggp[1]) * inv
    y1 = y1b[...] + jnp.dot(agg, wl[...].T,
                            preferred_element_type=jnp.float32)
    g1 = jax.nn.relu(_ln(y1, ng[...], nb[...]))
    y1_ref[...] = y1
    g1w_ref[...] = jnp.concatenate(
        [g1, jnp.zeros((_RB, _D - _DG), jnp.float32)], axis=1)
    y2b_ref[...] = x2_ref[...] + jnp.dot(
        g1, wr[...].T, preferred_element_type=jnp.float32) + bl[...]
    inv_ref[...] = inv


def _post_body(aggp, inv_ref, y1_ref, y2b, xres, wl, lg, lb, out_ref):
    agg = (aggp[0] + aggp[1]) * inv_ref[...]
    y2 = y2b[...] + jnp.dot(agg, wl[...].T,
                            preferred_element_type=jnp.float32)
    h = jnp.concatenate([y1_ref[...], y2], axis=1) + xres[...]
    out_ref[...] = _ln(h, lg[...], lb[...])


_EB = 2000
_NEB = (_E // 8) // _EB


def _edge_body(ea, m_ref, wb_ref, g1, b1, bt, g2, b2, out_ref):
    a = ea[...]
    m = m_ref[...]
    mu = jnp.dot(a, m, preferred_element_type=jnp.float32)
    d = a - mu
    var = jnp.dot(d * d, m, preferred_element_type=jnp.float32)
    h = _gelu(d * lax.rsqrt(var + _EPS) * g1[...] + b1[...])
    e = jnp.dot(h, wb_ref[...], preferred_element_type=jnp.float32) + bt[...] + a
    mu2 = jnp.dot(e, m, preferred_element_type=jnp.float32)
    d2 = e - mu2
    var2 = jnp.dot(d2 * d2, m, preferred_element_type=jnp.float32)
    out_ref[...] = d2 * lax.rsqrt(var2 + _EPS) * g2[...] + b2[...]


def _vec_spec(n):
    return pl.BlockSpec((n,), lambda i: (0,))


def _mat_spec(r, c):
    return pl.BlockSpec((r, c), lambda i: (0, 0))


def kernel(x, edge_index, edge_weights, edge_attr, pre_g, pre_b, ln_g, ln_b,
           b0_ng, b0_nb, b0_Wl, b0_bl, b0_Wr, b1_ng, b1_nb, b1_Wl, b1_bl,
           b1_Wr, et_ng, et_nb, et_W, et_b, en_g, en_b):
    src = edge_index[0]
    dst = edge_index[1]

    x2, h2w, y1b = pl.pallas_call(
        _pre_body,
        grid=(_NRB,),
        in_specs=[
            pl.BlockSpec((_RB, _D), lambda i: (i, 0)),
            _vec_spec(_D), _vec_spec(_D),
            _vec_spec(_DG), _vec_spec(_DG),
            _mat_spec(_DG, _DG), _vec_spec(_DG),
        ],
        out_specs=[
            pl.BlockSpec((_RB, _DG), lambda i: (i, 0)),
            pl.BlockSpec((_RB, _D), lambda i: (i, 0)),
            pl.BlockSpec((_RB, _DG), lambda i: (i, 0)),
        ],
        out_shape=[
            jax.ShapeDtypeStruct((_N, _DG), jnp.float32),
            jax.ShapeDtypeStruct((_N, _D), jnp.float32),
            jax.ShapeDtypeStruct((_N, _DG), jnp.float32),
        ],
    )(x, pre_g, pre_b, b0_ng, b0_nb, b0_Wr, b0_bl)

    agg0_p, cnt_p = _make_seg(True)(h2w.reshape(2 * _N, _DG), src, dst)
    cnt3 = cnt_p.reshape(_NC, _N, 1)

    y1, g1w, y2b, inv = pl.pallas_call(
        _mid_body,
        grid=(_NRB,),
        in_specs=[
            pl.BlockSpec((_NC, _RB, _DG), lambda i: (0, i, 0)),
            pl.BlockSpec((_NC, _RB, 1), lambda i: (0, i, 0)),
            pl.BlockSpec((_RB, _DG), lambda i: (i, 0)),
            pl.BlockSpec((_RB, _DG), lambda i: (i, 0)),
            _mat_spec(_DG, _DG),
            _vec_spec(_DG), _vec_spec(_DG),
            _mat_spec(_DG, _DG), _vec_spec(_DG),
        ],
        out_specs=[
            pl.BlockSpec((_RB, _DG), lambda i: (i, 0)),
            pl.BlockSpec((_RB, _D), lambda i: (i, 0)),
            pl.BlockSpec((_RB, _DG), lambda i: (i, 0)),
            pl.BlockSpec((_RB, 1), lambda i: (i, 0)),
        ],
        out_shape=[
            jax.ShapeDtypeStruct((_N, _DG), jnp.float32),
            jax.ShapeDtypeStruct((_N, _D), jnp.float32),
            jax.ShapeDtypeStruct((_N, _DG), jnp.float32),
            jax.ShapeDtypeStruct((_N, 1), jnp.float32),
        ],
    )(agg0_p, cnt3, y1b, x2, b0_Wl, b1_ng, b1_nb, b1_Wr, b1_bl)

    agg1_p = _make_seg(False)(g1w.reshape(2 * _N, _DG), src, dst)

    x_out = pl.pallas_call(
        _post_body,
        grid=(_NRB,),
        in_specs=[
            pl.BlockSpec((_NC, _RB, _DG), lambda i: (0, i, 0)),
            pl.BlockSpec((_RB, 1), lambda i: (i, 0)),
            pl.BlockSpec((_RB, _DG), lambda i: (i, 0)),
            pl.BlockSpec((_RB, _DG), lambda i: (i, 0)),
            pl.BlockSpec((_RB, _D), lambda i: (i, 0)),
            _mat_spec(_DG, _DG),
            _vec_spec(_D), _vec_spec(_D),
        ],
        out_specs=pl.BlockSpec((_RB, _D), lambda i: (i, 0)),
        out_shape=jax.ShapeDtypeStruct((_N, _D), jnp.float32),
    )(agg1_p, inv, y1, y2b, x, b1_Wl, ln_g, ln_b)

    eye8 = jnp.eye(8, dtype=jnp.float32)
    mmat = jnp.kron(eye8, jnp.full((_DE, _DE), 1.0 / _DE, jnp.float32))
    wbig = jnp.kron(eye8, et_W.T)
    ea2 = edge_attr.reshape(_E // 8, _D)
    e2 = pl.pallas_call(
        _edge_body,
        grid=(_NEB,),
        in_specs=[
            pl.BlockSpec((_EB, _D), lambda i: (i, 0)),
            _mat_spec(_D, _D), _mat_spec(_D, _D),
            _vec_spec(_D), _vec_spec(_D), _vec_spec(_D),
            _vec_spec(_D), _vec_spec(_D),
        ],
        out_specs=pl.BlockSpec((_EB, _D), lambda i: (i, 0)),
        out_shape=jax.ShapeDtypeStruct((_E // 8, _D), jnp.float32),
    )(ea2, mmat, wbig, jnp.tile(et_ng, 8), jnp.tile(et_nb, 8),
      jnp.tile(et_b, 8), jnp.tile(en_g, 8), jnp.tile(en_b, 8))
    e_out = e2.reshape(_E, _DE)

    return (x_out, edge_index, edge_weights, e_out)


# speedup vs baseline: 6.9161x; 1.0282x over previous
import functools

import jax
import jax.numpy as jnp
from jax import lax
from jax.experimental import pallas as pl
from jax.experimental.pallas import tpu as pltpu
from jax.experimental.pallas import tpu_sc as plsc

_N = 10000
_E = 320000
_D = 128
_DG = 64
_DE = 16
_EPS = 1e-5

_NC = 2
_NS = 16
_EPW = _E // (_NC * _NS)
_C = 128
_CT = _EPW - (_EPW // _C) * _C
_NCHUNK = _EPW // _C
_RPT = 624
_RPT_LAST = _N - _RPT * (_NS - 1)


def _ln(v, g, b):
    mu = jnp.mean(v, axis=-1, keepdims=True)
    d = v - mu
    var = jnp.mean(d * d, axis=-1, keepdims=True)
    return d * lax.rsqrt(var + _EPS) * g + b


def _gelu(v):
    return 0.5 * v * (1.0 + lax.erf(v * 0.7071067811865476))


def _seg_body(with_count, *refs):
    if with_count:
        (h_hbm, src_hbm, dst_hbm, agg_out, cnt_out,
         sidx, didx, rows, zbuf, ones_v, cbuf, acc_sh, cnt_sh,
         gsem0, gsem1) = refs
    else:
        (h_hbm, src_hbm, dst_hbm, agg_out,
         sidx, didx, rows, zbuf, acc_sh, gsem0, gsem1) = refs
    c = lax.axis_index("c")
    s = lax.axis_index("s")

    def _z0(i, _):
        for j in range(_DG // 16):
            zbuf[i, pl.ds(j * 16, 16)] = jnp.zeros((16,), jnp.float32)
        return 0
    lax.fori_loop(0, _RPT_LAST, _z0, 0)
    if with_count:
        for j in range(_RPT_LAST // 16):
            cbuf[pl.ds(j * 16, 16)] = jnp.zeros((16,), jnp.float32)
        for j in range(_C // 16):
            ones_v[pl.ds(j * 16, 16)] = jnp.ones((16,), jnp.float32)

    @pl.when(s < _NS - 1)
    def _():
        pltpu.sync_copy(zbuf.at[pl.ds(0, _RPT)], acc_sh.at[pl.ds(s * _RPT, _RPT)])
        if with_count:
            pltpu.sync_copy(cbuf.at[pl.ds(0, _RPT)],
                            cnt_sh.at[pl.ds(s * _RPT, _RPT)])

    @pl.when(s == _NS - 1)
    def _():
        pltpu.sync_copy(zbuf, acc_sh.at[pl.ds((_NS - 1) * _RPT, _RPT_LAST)])
        if with_count:
            pltpu.sync_copy(cbuf, cnt_sh.at[pl.ds((_NS - 1) * _RPT, _RPT_LAST)])

    ebase = (c * _NS + s) * _EPW
    pltpu.sync_copy(src_hbm.at[pl.ds(ebase, _EPW)], sidx)
    pltpu.sync_copy(dst_hbm.at[pl.ds(ebase, _EPW)], didx)

    def _dbl(i, _):
        v = sidx[pl.ds(i * 16, 16)]
        sidx[pl.ds(i * 16, 16)] = v + v
        return 0
    lax.fori_loop(0, _EPW // 16, _dbl, 0)

    plsc.subcore_barrier()

    def _sl(ref, k):
        return ref.at[pl.ds(k * _C, _C)]

    def _scat(k, b):
        pltpu.sync_copy(rows.at[b], acc_sh.at[_sl(didx, k)], add=True)
        if with_count:
            pltpu.sync_copy(ones_v, cnt_sh.at[_sl(didx, k)], add=True)

    pltpu.async_copy(h_hbm.at[_sl(sidx, 0)], rows.at[0], gsem0)

    def _wait(k, b, sem):
        pltpu.make_async_copy(h_hbm.at[_sl(sidx, k)], rows.at[b], sem).wait()

    def _pair(g, _):
        k0 = 2 * g
        pltpu.async_copy(h_hbm.at[_sl(sidx, k0 + 1)], rows.at[1], gsem1)
        _wait(k0, 0, gsem0)
        _scat(k0, 0)
        pltpu.async_copy(h_hbm.at[_sl(sidx, k0 + 2)], rows.at[0], gsem0)
        _wait(k0 + 1, 1, gsem1)
        _scat(k0 + 1, 1)
        return 0

    if _NCHUNK % 2 == 1:
        lax.fori_loop(0, (_NCHUNK - 1) // 2, _pair, 0)
        _wait(_NCHUNK - 1, 0, gsem0)
        _scat(_NCHUNK - 1, 0)
    else:
        lax.fori_loop(0, _NCHUNK // 2 - 1, _pair, 0)
        pltpu.async_copy(h_hbm.at[_sl(sidx, _NCHUNK - 1)], rows.at[1], gsem1)
        _wait(_NCHUNK - 2, 0, gsem0)
        _scat(_NCHUNK - 2, 0)
        _wait(_NCHUNK - 1, 1, gsem1)
        _scat(_NCHUNK - 1, 1)

    if _CT:
        tb = _NCHUNK * _C
        pltpu.async_copy(h_hbm.at[sidx.at[pl.ds(tb, _CT)]],
                         rows.at[0, pl.ds(0, _CT)], gsem0).wait()
        pltpu.sync_copy(rows.at[0, pl.ds(0, _CT)],
                        acc_sh.at[didx.at[pl.ds(tb, _CT)]], add=True)
        if with_count:
            pltpu.sync_copy(ones_v.at[pl.ds(0, _CT)],
                            cnt_sh.at[didx.at[pl.ds(tb, _CT)]], add=True)

    plsc.subcore_barrier()

    @pl.when(s < _NS - 1)
    def _():
        pltpu.sync_copy(acc_sh.at[pl.ds(s * _RPT, _RPT)], zbuf.at[pl.ds(0, _RPT)])
        pltpu.sync_copy(zbuf.at[pl.ds(0, _RPT)], agg_out.at[c, pl.ds(s * _RPT, _RPT)])
        if with_count:
            pltpu.sync_copy(cnt_sh.at[pl.ds(s * _RPT, _RPT)],
                            cbuf.at[pl.ds(0, _RPT)])
            pltpu.sync_copy(cbuf.at[pl.ds(0, _RPT)],
                            cnt_out.at[c, pl.ds(s * _RPT, _RPT)])

    @pl.when(s == _NS - 1)
    def _():
        pltpu.sync_copy(acc_sh.at[pl.ds((_NS - 1) * _RPT, _RPT_LAST)], zbuf)
        pltpu.sync_copy(zbuf, agg_out.at[c, pl.ds((_NS - 1) * _RPT, _RPT_LAST)])
        if with_count:
            pltpu.sync_copy(cnt_sh.at[pl.ds((_NS - 1) * _RPT, _RPT_LAST)], cbuf)
            pltpu.sync_copy(cbuf, cnt_out.at[c, pl.ds((_NS - 1) * _RPT, _RPT_LAST)])


def _make_seg(with_count):
    mesh = plsc.VectorSubcoreMesh(
        core_axis_name="c", subcore_axis_name="s",
        num_cores=_NC, num_subcores=_NS)
    out_type = [jax.ShapeDtypeStruct((_NC, _N, _DG), jnp.float32)]
    scratch = [
        pltpu.VMEM((_EPW,), jnp.int32),
        pltpu.VMEM((_EPW,), jnp.int32),
        pltpu.VMEM((2, _C, _DG), jnp.float32),
        pltpu.VMEM((_RPT_LAST, _DG), jnp.float32),
    ]
    if with_count:
        out_type.append(jax.ShapeDtypeStruct((_NC, _N), jnp.float32))
        scratch.append(pltpu.VMEM((_C,), jnp.float32))
        scratch.append(pltpu.VMEM((_RPT_LAST,), jnp.float32))
    scratch.append(pltpu.VMEM_SHARED((_N, _DG), jnp.float32))
    if with_count:
        scratch.append(pltpu.VMEM_SHARED((_N,), jnp.float32))
    scratch.append(pltpu.SemaphoreType.DMA)
    scratch.append(pltpu.SemaphoreType.DMA)
    return pl.kernel(
        functools.partial(_seg_body, with_count),
        out_type=tuple(out_type) if with_count else out_type[0],
        mesh=mesh,
        compiler_params=pltpu.CompilerParams(use_tc_tiling_on_sc=False),
        scratch_types=scratch,
    )


_RB = 1000
_NRB = _N // _RB


def _pre_body(x_ref, pg, pb, ng, nb, wr, bl,
              x2_ref, h2w_ref, y1b_ref):
    h = _gelu(_ln(x_ref[...], pg[...], pb[...]))
    x1 = h[:, :_DG]
    x2 = h[:, _DG:]
    h2 = jax.nn.relu(_ln(x2, ng[...], nb[...]))
    x2_ref[...] = x2
    h2w_ref[...] = jnp.concatenate(
        [h2, jnp.zeros((_RB, _D - _DG), jnp.float32)], axis=1)
    y1b_ref[...] = x1 + jnp.dot(h2, wr[...].T,
                                preferred_element_type=jnp.float32) + bl[...]


def _mid_body(aggp, cntp, y1b, x2_ref, wl, ng, nb, wr, bl,
              y1_ref, g1w_ref, y2b_ref, inv_ref):
    cnt = cntp[0] + cntp[1]
    inv = 1.0 / jnp.maximum(cnt, 1.0)
    agg = (aggp[0] + aggp[1]) * inv
    y1 = y1b[...] + jnp.dot(agg, wl[...].T,
                            preferred_element_type=jnp.float32)
    g1 = jax.nn.relu(_ln(y1, ng[...], nb[...]))
    y1_ref[...] = y1
    g1w_ref[...] = jnp.concatenate(
        [g1, jnp.zeros((_RB, _D - _DG), jnp.float32)], axis=1)
    y2b_ref[...] = x2_ref[...] + jnp.dot(
        g1, wr[...].T, preferred_element_type=jnp.float32) + bl[...]
    inv_ref[...] = inv


def _post_body(aggp, inv_ref, y1_ref, y2b, xres, wl, lg, lb, out_ref):
    agg = (aggp[0] + aggp[1]) * inv_ref[...]
    y2 = y2b[...] + jnp.dot(agg, wl[...].T,
                            preferred_element_type=jnp.float32)
    h = jnp.concatenate([y1_ref[...], y2], axis=1) + xres[...]
    out_ref[...] = _ln(h, lg[...], lb[...])


_EB = 2000
_NEB = (_E // 8) // _EB


def _edge_body(ea, m_ref, wb_ref, g1, b1, bt, g2, b2, out_ref):
    a = ea[...]
    m = m_ref[...]
    mu = jnp.dot(a, m, preferred_element_type=jnp.float32)
    d = a - mu
    var = jnp.dot(d * d, m, preferred_element_type=jnp.float32)
    h = _gelu(d * lax.rsqrt(var + _EPS) * g1[...] + b1[...])
    e = jnp.dot(h, wb_ref[...], preferred_element_type=jnp.float32) + bt[...] + a
    mu2 = jnp.dot(e, m, preferred_element_type=jnp.float32)
    d2 = e - mu2
    var2 = jnp.dot(d2 * d2, m, preferred_element_type=jnp.float32)
    out_ref[...] = d2 * lax.rsqrt(var2 + _EPS) * g2[...] + b2[...]


def _vec_spec(n):
    return pl.BlockSpec((n,), lambda i: (0,))


def _mat_spec(r, c):
    return pl.BlockSpec((r, c), lambda i: (0, 0))


def kernel(x, edge_index, edge_weights, edge_attr, pre_g, pre_b, ln_g, ln_b,
           b0_ng, b0_nb, b0_Wl, b0_bl, b0_Wr, b1_ng, b1_nb, b1_Wl, b1_bl,
           b1_Wr, et_ng, et_nb, et_W, et_b, en_g, en_b):
    src = edge_index[0]
    dst = edge_index[1]

    x2, h2w, y1b = pl.pallas_call(
        _pre_body,
        grid=(_NRB,),
        in_specs=[
            pl.BlockSpec((_RB, _D), lambda i: (i, 0)),
            _vec_spec(_D), _vec_spec(_D),
            _vec_spec(_DG), _vec_spec(_DG),
            _mat_spec(_DG, _DG), _vec_spec(_DG),
        ],
        out_specs=[
            pl.BlockSpec((_RB, _DG), lambda i: (i, 0)),
            pl.BlockSpec((_RB, _D), lambda i: (i, 0)),
            pl.BlockSpec((_RB, _DG), lambda i: (i, 0)),
        ],
        out_shape=[
            jax.ShapeDtypeStruct((_N, _DG), jnp.float32),
            jax.ShapeDtypeStruct((_N, _D), jnp.float32),
            jax.ShapeDtypeStruct((_N, _DG), jnp.float32),
        ],
    )(x, pre_g, pre_b, b0_ng, b0_nb, b0_Wr, b0_bl)

    agg0_p, cnt_p = _make_seg(True)(h2w.reshape(2 * _N, _DG), src, dst)
    cnt3 = cnt_p.reshape(_NC, _N, 1)

    y1, g1w, y2b, inv = pl.pallas_call(
        _mid_body,
        grid=(_NRB,),
        in_specs=[
            pl.BlockSpec((_NC, _RB, _DG), lambda i: (0, i, 0)),
            pl.BlockSpec((_NC, _RB, 1), lambda i: (0, i, 0)),
            pl.BlockSpec((_RB, _DG), lambda i: (i, 0)),
            pl.BlockSpec((_RB, _DG), lambda i: (i, 0)),
            _mat_spec(_DG, _DG),
            _vec_spec(_DG), _vec_spec(_DG),
            _mat_spec(_DG, _DG), _vec_spec(_DG),
        ],
        out_specs=[
            pl.BlockSpec((_RB, _DG), lambda i: (i, 0)),
            pl.BlockSpec((_RB, _D), lambda i: (i, 0)),
            pl.BlockSpec((_RB, _DG), lambda i: (i, 0)),
            pl.BlockSpec((_RB, 1), lambda i: (i, 0)),
        ],
        out_shape=[
            jax.ShapeDtypeStruct((_N, _DG), jnp.float32),
            jax.ShapeDtypeStruct((_N, _D), jnp.float32),
            jax.ShapeDtypeStruct((_N, _DG), jnp.float32),
            jax.ShapeDtypeStruct((_N, 1), jnp.float32),
        ],
    )(agg0_p, cnt3, y1b, x2, b0_Wl, b1_ng, b1_nb, b1_Wr, b1_bl)

    agg1_p = _make_seg(False)(g1w.reshape(2 * _N, _DG), src, dst)

    x_out = pl.pallas_call(
        _post_body,
        grid=(_NRB,),
        in_specs=[
            pl.BlockSpec((_NC, _RB, _DG), lambda i: (0, i, 0)),
            pl.BlockSpec((_RB, 1), lambda i: (i, 0)),
            pl.BlockSpec((_RB, _DG), lambda i: (i, 0)),
            pl.BlockSpec((_RB, _DG), lambda i: (i, 0)),
            pl.BlockSpec((_RB, _D), lambda i: (i, 0)),
            _mat_spec(_DG, _DG),
            _vec_spec(_D), _vec_spec(_D),
        ],
        out_specs=pl.BlockSpec((_RB, _D), lambda i: (i, 0)),
        out_shape=jax.ShapeDtypeStruct((_N, _D), jnp.float32),
    )(agg1_p, inv, y1, y2b, x, b1_Wl, ln_g, ln_b)

    eye8 = jnp.eye(8, dtype=jnp.float32)
    mmat = jnp.kron(eye8, jnp.full((_DE, _DE), 1.0 / _DE, jnp.float32))
    wbig = jnp.kron(eye8, et_W.T)
    ea2 = edge_attr.reshape(_E // 8, _D)
    e2 = pl.pallas_call(
        _edge_body,
        grid=(_NEB,),
        in_specs=[
            pl.BlockSpec((_EB, _D), lambda i: (i, 0)),
            _mat_spec(_D, _D), _mat_spec(_D, _D),
            _vec_spec(_D), _vec_spec(_D), _vec_spec(_D),
            _vec_spec(_D), _vec_spec(_D),
        ],
        out_specs=pl.BlockSpec((_EB, _D), lambda i: (i, 0)),
        out_shape=jax.ShapeDtypeStruct((_E // 8, _D), jnp.float32),
    )(ea2, mmat, wbig, jnp.tile(et_ng, 8), jnp.tile(et_nb, 8),
      jnp.tile(et_b, 8), jnp.tile(en_g, 8), jnp.tile(en_b, 8))
    e_out = e2.reshape(_E, _DE)

    return (x_out, edge_index, edge_weights, e_out)
